# Initial kernel scaffold; baseline (speedup 1.0000x reference)
import jax
import jax.numpy as jnp
from jax.experimental import pallas as pl


def kernel(pred_cls, pred_reg, pred_cls2, pred_reg2, pred_cls3, pred_reg3, pred_cls4, pred_reg4, anchors, targets):
    raise NotImplementedError("write your pallas kernel here")


# speedup vs baseline: 4.9825x; 4.9825x over previous
import jax
import jax.numpy as jnp
from jax.experimental import pallas as pl
from jax.experimental.pallas import tpu as pltpu

_C = 30
_H = 64
_W = 64
_PER_HEAD = _C * _H * _W
_K = 100
_TH = 0.05
_BIG = 1 << 30


def _pp_kernel(cls1, cls2, cls3, cls4, reg1, reg2, reg3, reg4, anch,
               det_out, val_out, g_out,
               sa, m1, g1, rsc, sem):
    b = pl.program_id(0)

    q_i = jax.lax.broadcasted_iota(jnp.int32, (6, _H, _W), 0)
    y_i = jax.lax.broadcasted_iota(jnp.int32, (6, _H, _W), 1)
    x_i = jax.lax.broadcasted_iota(jnp.int32, (6, _H, _W), 2)
    g0 = ((y_i * _W + x_i) * _C + q_i).astype(jnp.float32)
    for h, cref in enumerate((cls1, cls2, cls3, cls4)):
        for qc in range(0, _C, 6):
            s = jax.nn.sigmoid(cref[0, qc:qc + 6])
            s = jnp.where(s > _TH, s, 0.0)
            sa[h * _C + qc: h * _C + qc + 6] = s
            g = g0 + float(qc + h * _PER_HEAD)
            mx = jnp.max(s, axis=2)
            gm = jnp.min(jnp.where(s == mx[:, :, None], g, float(_BIG)),
                         axis=2)
            m1[h * _C + qc: h * _C + qc + 6, :] = mx
            g1[h * _C + qc: h * _C + qc + 6, :] = gm

    xiota = jax.lax.broadcasted_iota(jnp.int32, (1, _W), 1)
    liota = jax.lax.broadcasted_iota(jnp.int32, (128, 1), 0)

    def body(k, carry):
        vals, gacc, wv, hv, cxv, cyv, xoffv = carry
        m = jnp.max(m1[:, :])
        gstar = jnp.min(jnp.where(m1[:, :] == m, g1[:, :], float(_BIG)))
        gstar = gstar.astype(jnp.int32)
        hh = gstar // _PER_HEAD
        rem = gstar - hh * _PER_HEAD
        p = rem // _C
        q = rem - p * _C
        y = p // _W
        x = p - y * _W
        row = hh * _C + q

        rowv = sa[pl.ds(row, 1), pl.ds(y, 1), :]
        rowv = jnp.where(xiota[None] == x, -1.0, rowv)
        sa[pl.ds(row, 1), pl.ds(y, 1), :] = rowv
        grow = ((y * _W + xiota) * _C + q + hh * _PER_HEAD
                ).astype(jnp.float32)
        mrow = jnp.max(rowv)
        grm = jnp.min(jnp.where(rowv[0] == mrow, grow, float(_BIG)))
        mr = m1[pl.ds(row, 1), :]
        m1[pl.ds(row, 1), :] = jnp.where(xiota == y, mrow, mr)
        gr = g1[pl.ds(row, 1), :]
        g1[pl.ds(row, 1), :] = jnp.where(xiota == y, grm, gr)

        arow = anch[0, pl.ds(p, 1), :]
        a0 = arow[0, 0]
        a1 = arow[0, 1]
        a2 = arow[0, 2]
        a3 = arow[0, 3]

        qs = q * 16
        pa = (p // 128) * 128
        xo = p - pa

        def start(rref):
            pltpu.make_async_copy(
                rref.at[b, pl.ds(qs, 16), pl.ds(pa, 128)],
                rsc.at[k],
                sem,
            ).start()

        jax.lax.switch(hh, [
            lambda: start(reg1),
            lambda: start(reg2),
            lambda: start(reg3),
            lambda: start(reg4),
        ])

        sel = liota == k
        vals = jnp.where(sel, m, vals)
        gacc = jnp.where(sel, gstar, gacc)
        wv = jnp.where(sel, a2 - a0, wv)
        hv = jnp.where(sel, a3 - a1, hv)
        cxv = jnp.where(sel, (a0 + a2) * 0.5, cxv)
        cyv = jnp.where(sel, (a1 + a3) * 0.5, cyv)
        xoffv = jnp.where(sel, xo, xoffv)
        return vals, gacc, wv, hv, cxv, cyv, xoffv

    zf = jnp.zeros((128, 1), jnp.float32)
    zi = jnp.zeros((128, 1), jnp.int32)
    vals, gacc, wv, hv, cxv, cyv, xoffv = jax.lax.fori_loop(
        0, _K, body, (zf, zi, zf, zf, zf, zf, zi))

    def wbody(_, c):
        pltpu.make_async_copy(
            reg1.at[0, pl.ds(0, 16), pl.ds(0, 128)],
            rsc.at[0],
            sem,
        ).wait()
        return c
    jax.lax.fori_loop(0, _K, wbody, 0)

    oiota = jax.lax.broadcasted_iota(jnp.int32, (128, 16, 128), 2)
    rv = jnp.sum(jnp.where(oiota == xoffv[:, :, None], rsc[:, :, :], 0.0),
                 axis=2)
    jio = jax.lax.broadcasted_iota(jnp.int32, (128, 16), 1)
    scale = jnp.where(jio < 8, wv, hv)
    off = jnp.where(jio < 8, cxv, cyv)
    det_out[0] = rv * scale + off
    val_out[0] = jnp.sqrt(vals + 1e-12)
    g_out[0] = gacc


def kernel(pred_cls, pred_reg, pred_cls2, pred_reg2, pred_cls3, pred_reg3,
           pred_cls4, pred_reg4, anchors, targets):
    del targets
    n = pred_cls.shape[0]
    cls_spec = pl.BlockSpec((1, _C, _H, _W), lambda i: (i, 0, 0, 0))
    reg_spec = pl.BlockSpec(memory_space=pl.ANY)
    anch_spec = pl.BlockSpec((1, _H * _W, 4), lambda i: (i, 0, 0))
    det, val, g = pl.pallas_call(
        _pp_kernel,
        grid=(n,),
        in_specs=[cls_spec, cls_spec, cls_spec, cls_spec,
                  reg_spec, reg_spec, reg_spec, reg_spec, anch_spec],
        out_specs=[pl.BlockSpec((1, 128, 16), lambda i: (i, 0, 0)),
                   pl.BlockSpec((1, 128, 1), lambda i: (i, 0, 0)),
                   pl.BlockSpec((1, 128, 1), lambda i: (i, 0, 0))],
        out_shape=[jax.ShapeDtypeStruct((n, 128, 16), jnp.float32),
                   jax.ShapeDtypeStruct((n, 128, 1), jnp.float32),
                   jax.ShapeDtypeStruct((n, 128, 1), jnp.int32)],
        scratch_shapes=[pltpu.VMEM((4 * _C, _H, _W), jnp.float32),
                        pltpu.VMEM((4 * _C, _H), jnp.float32),
                        pltpu.VMEM((4 * _C, _H), jnp.float32),
                        pltpu.VMEM((128, 16, 128), jnp.float32),
                        pltpu.SemaphoreType.DMA],
    )(pred_cls, pred_cls2, pred_cls3, pred_cls4,
      pred_reg.reshape(n, _C * 16, _H * _W),
      pred_reg2.reshape(n, _C * 16, _H * _W),
      pred_reg3.reshape(n, _C * 16, _H * _W),
      pred_reg4.reshape(n, _C * 16, _H * _W), anchors)

    out = jnp.concatenate([det[:, :_K, :], val[:, :_K, :]], axis=-1)
    lb = (g[:, :_K, 0] % _PER_HEAD) % _C + 1
    return out, lb
